# baseline (speedup 1.0000x reference)
import functools

import jax
import jax.numpy as jnp
from jax import lax
from jax.experimental import pallas as pl
from jax.experimental.pallas import tpu as pltpu
from jax.experimental.pallas import tpu_sc as plsc

VOCAB = 1002
N_D = 64
BATCH = 4096
HIST = 200

NC = 2
NS = 16
NW = NC * NS

B = BATCH * HIST
BPW = B // NW
GROUP = 512
SUB = 128
NSUB = GROUP // SUB
K = 5
NBODY = BPW // (GROUP * K)


def _emb_body(idx_hbm, table_hbm, out_hbm,
              idx_v, rows_v, gsem0, gsem1, ssem0, ssem1):
    wid = lax.axis_index("s") * NC + lax.axis_index("c")
    base = wid * BPW
    gsems = (gsem0, gsem1)
    ssems = (ssem0, ssem1)

    def fire(row0, b):
        pltpu.sync_copy(idx_hbm.at[pl.ds(row0, GROUP)], idx_v.at[b])
        return [
            pltpu.async_copy(
                table_hbm.at[idx_v.at[b, pl.ds(j * SUB, SUB)]],
                rows_v.at[b, pl.ds(j * SUB, SUB)],
                gsems[b],
            )
            for j in range(NSUB)
        ]

    def scatter(row0, b):
        return pltpu.async_copy(
            rows_v.at[b], out_hbm.at[pl.ds(row0, GROUP)], ssems[b]
        )

    def body(i, carry):
        blk0 = base + i * (GROUP * K)
        g_descs = {}
        s_descs = {}
        for k in range(K):
            b = k % 2
            row0 = blk0 + k * GROUP
            if k >= 2:
                s_descs[k - 2].wait()
            g_descs[k] = fire(row0, b)
            if k >= 1:
                for cp in g_descs[k - 1]:
                    cp.wait()
                s_descs[k - 1] = scatter(blk0 + (k - 1) * GROUP, (k - 1) % 2)
        for cp in g_descs[K - 1]:
            cp.wait()
        s_descs[K - 1] = scatter(blk0 + (K - 1) * GROUP, (K - 1) % 2)
        s_descs[K - 2].wait()
        s_descs[K - 1].wait()
        return carry

    lax.fori_loop(0, NBODY, body, 0)


@jax.jit
def _embedding_sc(idx_flat, table):
    mesh = plsc.VectorSubcoreMesh(
        core_axis_name="c", subcore_axis_name="s",
        num_cores=NC, num_subcores=NS,
    )
    f = functools.partial(
        pl.kernel,
        out_type=jax.ShapeDtypeStruct((B, N_D), jnp.float32),
        mesh=mesh,
        scratch_types=[
            pltpu.VMEM((2, GROUP), jnp.int32),
            pltpu.VMEM((2, GROUP, N_D), jnp.float32),
            pltpu.SemaphoreType.DMA,
            pltpu.SemaphoreType.DMA,
            pltpu.SemaphoreType.DMA,
            pltpu.SemaphoreType.DMA,
        ],
        compiler_params=pltpu.CompilerParams(use_tc_tiling_on_sc=False),
    )(_emb_body)
    return f(idx_flat, table)


def kernel(input, table):
    idx_flat = input.reshape(-1).astype(jnp.int32)
    out = _embedding_sc(idx_flat, table)
    return out.reshape(BATCH, HIST, N_D)


# speedup vs baseline: 1.1621x; 1.1621x over previous
import functools

import jax
import jax.numpy as jnp
from jax import lax
from jax.experimental import pallas as pl
from jax.experimental.pallas import tpu as pltpu
from jax.experimental.pallas import tpu_sc as plsc

VOCAB = 1002
N_D = 64
BATCH = 4096
HIST = 200

NC = 2
NS = 16
NW = NC * NS

NB = BATCH // NW
K = 8
NBODY = NB // K
RUNROLL = 8
LANES = 16
NCH = N_D // LANES


def _emb_body(idx_hbm, table_hbm, out_hbm, table_v, idx_v, out_v, sem0, sem1):
    wid = lax.axis_index("s") * NC + lax.axis_index("c")
    base = wid * NB
    sems = (sem0, sem1)

    pltpu.sync_copy(table_hbm, table_v)

    def compute_group(k, b):
        def emit_row(row, src):
            for c in range(NCH):
                out_v[b, row, pl.ds(c * LANES, LANES)] = (
                    table_v[pl.ds(src + c * LANES, LANES)]
                )

        def rows16(i, carry):
            iv = idx_v[pl.ds(k * HIST + i * LANES, LANES)]
            for r in range(LANES):
                emit_row(i * LANES + r, iv[r] * N_D)
            return carry

        lax.fori_loop(0, (HIST - 8) // LANES, rows16, 0)
        iv = idx_v[pl.ds(k * HIST + HIST - LANES, LANES)]
        for r in range(8, LANES):
            emit_row(HIST - LANES + r, iv[r] * N_D)

    def body(i, carry):
        blk = base + i * K
        pltpu.sync_copy(idx_hbm.at[pl.ds(blk * HIST, K * HIST)], idx_v)
        descs = {}
        for k in range(K):
            b = k % 2
            if k >= 2:
                descs[k - 2].wait()
            compute_group(k, b)
            descs[k] = pltpu.async_copy(out_v.at[b], out_hbm.at[blk + k], sems[b])
        descs[K - 2].wait()
        descs[K - 1].wait()
        return carry

    lax.fori_loop(0, NBODY, body, 0)


@jax.jit
def _embedding_sc(idx_flat, table_flat):
    mesh = plsc.VectorSubcoreMesh(
        core_axis_name="c", subcore_axis_name="s",
        num_cores=NC, num_subcores=NS,
    )
    f = functools.partial(
        pl.kernel,
        out_type=jax.ShapeDtypeStruct((BATCH, HIST, N_D), jnp.float32),
        mesh=mesh,
        scratch_types=[
            pltpu.VMEM((VOCAB * N_D,), jnp.float32),
            pltpu.VMEM((K * HIST,), jnp.int32),
            pltpu.VMEM((2, HIST, N_D), jnp.float32),
            pltpu.SemaphoreType.DMA,
            pltpu.SemaphoreType.DMA,
        ],
        compiler_params=pltpu.CompilerParams(use_tc_tiling_on_sc=True),
    )(_emb_body)
    return f(idx_flat, table_flat)


def kernel(input, table):
    idx_flat = input.reshape(-1).astype(jnp.int32)
    return _embedding_sc(idx_flat, table.reshape(-1))
